# baseline (speedup 1.0000x reference)
import jax
import jax.numpy as jnp
from jax import lax
from jax.experimental import pallas as pl
from jax.experimental.pallas import tpu as pltpu
from jax.experimental.pallas import tpu_sc as plsc

F = 26
V = 100000
D = 32
B = 16384
NC, NS, L = 2, 16, 16
NW = NC * NS
ROWS = F * D
RPW = ROWS // NW
PC = 2048
NP = B // PC


def _body(tt2, idxt, out2, row_v, ibig, obuf, rsem, isem, *osem):
    w = lax.axis_index("s") * NC + lax.axis_index("c")

    def do_row(u, _):
        ft = w * RPW + u
        f = lax.div(ft, D)

        pltpu.async_copy(tt2.at[ft], row_v, rsem)

        @pl.when(jnp.logical_or(u == 0, lax.rem(ft, D) == 0))
        def _():
            pltpu.async_copy(idxt.at[f], ibig, isem)
            pltpu.make_async_copy(idxt.at[f], ibig, isem).wait()

        pltpu.make_async_copy(tt2.at[ft], row_v, rsem).wait()

        def do_pair(p2, _):
            for pb in range(2):
                p = p2 * 2 + pb

                @pl.when(jnp.logical_or(p2 >= 1, u > 0))
                def _():
                    pltpu.make_async_copy(
                        obuf.at[pb], out2.at[ft, pl.ds(p * PC, PC)], osem[pb]
                    ).wait()

                def extract(q, _):
                    for t in range(8):
                        o = q * 8 * L + t * L
                        obuf[pb, pl.ds(o, L)] = plsc.load_gather(
                            row_v, [ibig[pl.ds(p * PC + o, L)]]
                        )
                    return 0

                lax.fori_loop(0, PC // (8 * L), extract, 0)
                pltpu.async_copy(
                    obuf.at[pb], out2.at[ft, pl.ds(p * PC, PC)], osem[pb]
                )
            return 0

        lax.fori_loop(0, NP // 2, do_pair, 0)
        return 0

    lax.fori_loop(0, RPW, do_row, 0)

    ftl = w * RPW + RPW - 1
    for pb in range(2):
        pltpu.make_async_copy(
            obuf.at[pb], out2.at[ftl, pl.ds(pb * PC, PC)], osem[pb]
        ).wait()


@jax.jit
def kernel(sparse_inputs, tables):
    tt2 = tables.transpose(0, 2, 1).reshape(ROWS, V)
    idxt = sparse_inputs.T

    mesh = plsc.VectorSubcoreMesh(
        core_axis_name="c", subcore_axis_name="s", num_cores=NC, num_subcores=NS
    )
    out2 = pl.kernel(
        _body,
        out_type=jax.ShapeDtypeStruct((ROWS, B), jnp.float32),
        mesh=mesh,
        compiler_params=pltpu.CompilerParams(
            use_tc_tiling_on_sc=True,
            needs_layout_passes=False,
            disable_bounds_checks=True,
        ),
        scratch_types=(
            [
                pltpu.VMEM((V,), jnp.float32),
                pltpu.VMEM((B,), jnp.int32),
                pltpu.VMEM((2, PC), jnp.float32),
                pltpu.SemaphoreType.DMA,
                pltpu.SemaphoreType.DMA,
            ]
            + [pltpu.SemaphoreType.DMA] * 2
        ),
    )(tt2, idxt)
    return out2.reshape(F, D, B).transpose(2, 0, 1)


# speedup vs baseline: 2.0372x; 2.0372x over previous
import jax
import jax.numpy as jnp
from jax import lax
from jax.experimental import pallas as pl
from jax.experimental.pallas import tpu as pltpu
from jax.experimental.pallas import tpu_sc as plsc

F = 26
V = 100000
D = 32
B = 16384
NC, NS, L = 2, 16, 16
NW = NC * NS
ROWS = F * D
RPW = ROWS // NW
PC = 2048
NP = B // PC


def _body(tt2, idxt, out2, row_v, ibig, obuf, rsem, isem, *osem):
    w = lax.axis_index("s") * NC + lax.axis_index("c")

    def do_row(u, _):
        ft = w * RPW + u
        f = lax.div(ft, D)

        pltpu.async_copy(tt2.at[ft], row_v, rsem)

        @pl.when(jnp.logical_or(u == 0, lax.rem(ft, D) == 0))
        def _():
            pltpu.async_copy(idxt.at[f], ibig, isem)
            pltpu.make_async_copy(idxt.at[f], ibig, isem).wait()

        pltpu.make_async_copy(tt2.at[ft], row_v, rsem).wait()

        def do_pair(p2, _):
            for pb in range(2):
                p = p2 * 2 + pb

                @pl.when(jnp.logical_or(p2 >= 1, u > 0))
                def _():
                    pltpu.make_async_copy(
                        obuf.at[pb], out2.at[ft, pl.ds(p * PC, PC)], osem[pb]
                    ).wait()

                @plsc.parallel_loop(0, PC, step=L, unroll=8)
                def _(o):
                    obuf[pb, pl.ds(o, L)] = plsc.load_gather(
                        row_v, [ibig[pl.ds(p * PC + o, L)]]
                    )
                pltpu.async_copy(
                    obuf.at[pb], out2.at[ft, pl.ds(p * PC, PC)], osem[pb]
                )
            return 0

        lax.fori_loop(0, NP // 2, do_pair, 0)
        return 0

    lax.fori_loop(0, RPW, do_row, 0)

    ftl = w * RPW + RPW - 1
    for pb in range(2):
        pltpu.make_async_copy(
            obuf.at[pb], out2.at[ftl, pl.ds(pb * PC, PC)], osem[pb]
        ).wait()


@jax.jit
def kernel(sparse_inputs, tables):
    tt2 = tables.transpose(0, 2, 1).reshape(ROWS, V)
    idxt = sparse_inputs.T

    mesh = plsc.VectorSubcoreMesh(
        core_axis_name="c", subcore_axis_name="s", num_cores=NC, num_subcores=NS
    )
    out2 = pl.kernel(
        _body,
        out_type=jax.ShapeDtypeStruct((ROWS, B), jnp.float32),
        mesh=mesh,
        compiler_params=pltpu.CompilerParams(
            use_tc_tiling_on_sc=True,
            needs_layout_passes=False,
            disable_bounds_checks=True,
        ),
        scratch_types=(
            [
                pltpu.VMEM((V,), jnp.float32),
                pltpu.VMEM((B,), jnp.int32),
                pltpu.VMEM((2, PC), jnp.float32),
                pltpu.SemaphoreType.DMA,
                pltpu.SemaphoreType.DMA,
            ]
            + [pltpu.SemaphoreType.DMA] * 2
        ),
    )(tt2, idxt)
    return out2.reshape(F, D, B).transpose(2, 0, 1)


# speedup vs baseline: 2.0449x; 1.0038x over previous
import jax
import jax.numpy as jnp
from jax import lax
from jax.experimental import pallas as pl
from jax.experimental.pallas import tpu as pltpu
from jax.experimental.pallas import tpu_sc as plsc

F = 26
V = 100000
D = 32
B = 16384
NC, NS, L = 2, 16, 16
NW = NC * NS
ROWS = F * D
RPW = ROWS // NW
PC = 2048
NP = B // PC


def _body(tt2, idxt, out2, row_v, ibig, obuf, rsem, isem, *osem):
    w = lax.axis_index("s") * NC + lax.axis_index("c")

    def do_row(u, _):
        ft = w * RPW + u
        f = lax.div(ft, D)

        pltpu.async_copy(tt2.at[ft], row_v, rsem)

        @pl.when(jnp.logical_or(u == 0, lax.rem(ft, D) == 0))
        def _():
            pltpu.async_copy(idxt.at[f], ibig, isem)
            pltpu.make_async_copy(idxt.at[f], ibig, isem).wait()

        pltpu.make_async_copy(tt2.at[ft], row_v, rsem).wait()

        def do_pair(p2, _):
            for pb in range(2):
                p = p2 * 2 + pb

                @pl.when(jnp.logical_or(p2 >= 1, u > 0))
                def _():
                    pltpu.make_async_copy(
                        obuf.at[pb], out2.at[ft, pl.ds(p * PC, PC)], osem[pb]
                    ).wait()

                @plsc.parallel_loop(0, PC, step=L, unroll=16)
                def _(o):
                    obuf[pb, pl.ds(o, L)] = plsc.load_gather(
                        row_v, [ibig[pl.ds(p * PC + o, L)]]
                    )
                pltpu.async_copy(
                    obuf.at[pb], out2.at[ft, pl.ds(p * PC, PC)], osem[pb]
                )
            return 0

        lax.fori_loop(0, NP // 2, do_pair, 0)
        return 0

    lax.fori_loop(0, RPW, do_row, 0)

    ftl = w * RPW + RPW - 1
    for pb in range(2):
        pltpu.make_async_copy(
            obuf.at[pb], out2.at[ftl, pl.ds(pb * PC, PC)], osem[pb]
        ).wait()


@jax.jit
def kernel(sparse_inputs, tables):
    tt2 = tables.transpose(0, 2, 1).reshape(ROWS, V)
    idxt = sparse_inputs.T

    mesh = plsc.VectorSubcoreMesh(
        core_axis_name="c", subcore_axis_name="s", num_cores=NC, num_subcores=NS
    )
    out2 = pl.kernel(
        _body,
        out_type=jax.ShapeDtypeStruct((ROWS, B), jnp.float32),
        mesh=mesh,
        compiler_params=pltpu.CompilerParams(
            use_tc_tiling_on_sc=True,
            needs_layout_passes=False,
            disable_bounds_checks=True,
        ),
        scratch_types=(
            [
                pltpu.VMEM((V,), jnp.float32),
                pltpu.VMEM((B,), jnp.int32),
                pltpu.VMEM((2, PC), jnp.float32),
                pltpu.SemaphoreType.DMA,
                pltpu.SemaphoreType.DMA,
            ]
            + [pltpu.SemaphoreType.DMA] * 2
        ),
    )(tt2, idxt)
    return out2.reshape(F, D, B).transpose(2, 0, 1)
